# baseline (speedup 1.0000x reference)
import jax
import jax.numpy as jnp
from jax import lax
from jax.experimental import pallas as pl
from jax.experimental.pallas import tpu as pltpu
from jax.experimental.pallas import tpu_sc as plsc

NC, NS = 2, 16
NW = NC * NS
B, L = 16384, 20
R = B * L
CHUNK = 128
NBUF = 4
BT = B // CHUNK
QPW = BT // NW
PER_W = QPW * L
ROUNDS_PER_Q = L // NBUF

D_ID = 32
D_CAT = 16
D_OUT = D_ID + 3 * D_CAT
NT = D_OUT // 8
TILE = 8 * CHUNK
OUT_FLAT = L * NT * BT * TILE


def _emb_body(nid, sec, reg, ven, id_t, sec_t, reg_t, ven_t, out,
              slab, idx_x, id_r, sec_r, reg_r, ven_r, stage,
              sem_i, sem_g, sem_w):
    wid = lax.axis_index("s") * NC + lax.axis_index("c")
    iota16 = lax.iota(jnp.int32, 16)

    def load_slabs(bt):
        rows = pl.ds(bt * CHUNK * L, CHUNK * L)
        pltpu.sync_copy(nid.at[rows], slab.at[0])
        pltpu.sync_copy(sec.at[rows], slab.at[1])
        pltpu.sync_copy(reg.at[rows], slab.at[2])
        pltpu.sync_copy(ven.at[rows], slab.at[3])

    iota_l = iota16 * L

    def extract_indices():
        def per_l(l, carry):
            for a in range(4):
                for m in range(8):
                    idx_x[a, l, pl.ds(m * 16, 16)] = plsc.load_gather(
                        slab.at[a], [iota_l + (m * 16 * L + l)])
            return carry

        lax.fori_loop(0, L, per_l, 0)

    def gather_copies(l, b):
        return [
            pltpu.make_async_copy(id_t.at[idx_x.at[0, l]], id_r.at[b],
                                  sem_g.at[b]),
            pltpu.make_async_copy(sec_t.at[idx_x.at[1, l]], sec_r.at[b],
                                  sem_g.at[b]),
            pltpu.make_async_copy(reg_t.at[idx_x.at[2, l]], reg_r.at[b],
                                  sem_g.at[b]),
            pltpu.make_async_copy(ven_t.at[idx_x.at[3, l]], ven_r.at[b],
                                  sem_g.at[b]),
        ]

    def tile_src(b, t):
        if t < 4:
            return id_r.at[b], t * 8
        if t < 6:
            return sec_r.at[b], (t - 4) * 8
        if t < 8:
            return reg_r.at[b], (t - 6) * 8
        return ven_r.at[b], (t - 8) * 8

    iota128 = iota16 * CHUNK

    def transpose(b):
        groups = [
            (id_r, 0, 0), (id_r, 16, 16 * CHUNK),
            (sec_r, 0, D_ID * CHUNK), (reg_r, 0, (D_ID + 16) * CHUNK),
            (ven_r, 0, (D_ID + 32) * CHUNK),
        ]

        def row(br, carry):
            for ref, sl, gbase in groups:
                plsc.store_scatter(stage.at[b], [iota128 + (br + gbase)],
                                   ref[b, br, pl.ds(sl, 16)])
            return carry

        lax.fori_loop(0, CHUNK, row, 0, unroll=4)

    def write_copies(l, bt, b):
        return [
            pltpu.make_async_copy(
                stage.at[b, pl.ds(t * TILE, TILE)],
                out.at[pl.ds(((l * NT + t) * BT + bt) * TILE, TILE)],
                sem_w.at[b])
            for t in range(NT)
        ]

    def outer(g, carry):
        q = g // ROUNDS_PER_Q
        l0 = (g % ROUNDS_PER_Q) * NBUF
        bt = wid * QPW + q

        @pl.when(g % ROUNDS_PER_Q == 0)
        def _():
            load_slabs(bt)
            extract_indices()

        for b in range(NBUF):
            @pl.when(g > 0)
            def _():
                for cp in write_copies(l0 + b, bt, b):
                    cp.wait()

        for b in range(NBUF):
            for cp in gather_copies(l0 + b, b):
                cp.start()

        for b in range(NBUF):
            for cp in gather_copies(l0 + b, b):
                cp.wait()
            transpose(b)
            for cp in write_copies(l0 + b, bt, b):
                cp.start()

        return carry

    lax.fori_loop(0, PER_W // NBUF, outer, 0, unroll=False)

    for b in range(NBUF):
        for cp in write_copies(0, wid * QPW, b):
            cp.wait()


def kernel(node_ids, cat_sector, cat_region, cat_venue,
           id_table, sector_table, region_table, venue_table):
    call = pl.kernel(
        _emb_body,
        out_type=jax.ShapeDtypeStruct((OUT_FLAT,), jnp.float32),
        mesh=plsc.VectorSubcoreMesh(
            core_axis_name="c", subcore_axis_name="s",
            num_cores=NC, num_subcores=NS),
        scratch_types=[
            pltpu.VMEM((4, CHUNK * L), jnp.int32),
            pltpu.VMEM((4, L, CHUNK), jnp.int32),
            pltpu.VMEM((NBUF, CHUNK, D_ID), jnp.float32),
            pltpu.VMEM((NBUF, CHUNK, D_CAT), jnp.float32),
            pltpu.VMEM((NBUF, CHUNK, D_CAT), jnp.float32),
            pltpu.VMEM((NBUF, CHUNK, D_CAT), jnp.float32),
            pltpu.VMEM((NBUF, NT * TILE), jnp.float32),
            pltpu.SemaphoreType.DMA((NBUF,)),
            pltpu.SemaphoreType.DMA((NBUF,)),
            pltpu.SemaphoreType.DMA((NBUF,)),
        ],
        compiler_params=pltpu.CompilerParams(use_tc_tiling_on_sc=False,
                                             needs_layout_passes=False),
    )
    flat = call(node_ids.reshape(-1).astype(jnp.int32),
                cat_sector.reshape(-1).astype(jnp.int32),
                cat_region.reshape(-1).astype(jnp.int32),
                cat_venue.reshape(-1).astype(jnp.int32),
                id_table, sector_table, region_table, venue_table)
    return (flat.reshape(L, NT, BT, 8, CHUNK)
                .transpose(2, 4, 0, 1, 3)
                .reshape(B, L, D_OUT))


# speedup vs baseline: 1.0616x; 1.0616x over previous
import jax
import jax.numpy as jnp
from jax import lax
from jax.experimental import pallas as pl
from jax.experimental.pallas import tpu as pltpu
from jax.experimental.pallas import tpu_sc as plsc

NC, NS = 2, 16
NW = NC * NS
B, L = 16384, 20
R = B * L
CHUNK = 128
NBUF = 4
N_CHUNKS = L * (B // CHUNK)
PER_W = N_CHUNKS // NW

D_ID = 32
D_CAT = 16
D_OUT = D_ID + 3 * D_CAT
NT = D_OUT // 8
BT = B // CHUNK
TILE = 8 * CHUNK
OUT_FLAT = L * NT * BT * TILE


def _emb_body(nid, sec, reg, ven, id_t, sec_t, reg_t, ven_t, out,
              idx_v, id_r, sec_r, reg_r, ven_r, stage, sem_i, sem_g, sem_w):
    wid = lax.axis_index("s") * NC + lax.axis_index("c")
    c_base = wid * PER_W
    iota16 = lax.iota(jnp.int32, 16)

    def idx_off(c):
        l = c // BT
        bt = c % BT
        return l * B + bt * CHUNK

    def idx_copies(c, b):
        o = idx_off(c)
        return [
            pltpu.make_async_copy(nid.at[pl.ds(o, CHUNK)], idx_v.at[b, 0],
                                  sem_i.at[b]),
            pltpu.make_async_copy(sec.at[pl.ds(o, CHUNK)], idx_v.at[b, 1],
                                  sem_i.at[b]),
            pltpu.make_async_copy(reg.at[pl.ds(o, CHUNK)], idx_v.at[b, 2],
                                  sem_i.at[b]),
            pltpu.make_async_copy(ven.at[pl.ds(o, CHUNK)], idx_v.at[b, 3],
                                  sem_i.at[b]),
        ]

    def gather_copies(b):
        return [
            pltpu.make_async_copy(id_t.at[idx_v.at[b, 0]], id_r.at[b],
                                  sem_g.at[b]),
            pltpu.make_async_copy(sec_t.at[idx_v.at[b, 1]], sec_r.at[b],
                                  sem_g.at[b]),
            pltpu.make_async_copy(reg_t.at[idx_v.at[b, 2]], reg_r.at[b],
                                  sem_g.at[b]),
            pltpu.make_async_copy(ven_t.at[idx_v.at[b, 3]], ven_r.at[b],
                                  sem_g.at[b]),
        ]

    def tile_src(b, t):
        if t < 4:
            return id_r.at[b], t * 8
        if t < 6:
            return sec_r.at[b], (t - 4) * 8
        if t < 8:
            return reg_r.at[b], (t - 6) * 8
        return ven_r.at[b], (t - 8) * 8

    def transpose(b):
        def grp(j, carry):
            dr = j // 8
            br0 = (j % 8) * 16
            rows = br0 + iota16
            for t in range(NT):
                src, d0 = tile_src(b, t)
                cols = jnp.full((16,), d0 + dr, jnp.int32)
                stage[b, t, pl.ds(dr * CHUNK + br0, 16)] = (
                    plsc.load_gather(src, [rows, cols]))
            return carry

        lax.fori_loop(0, 64, grp, 0, unroll=2)

    def write_copies(c, b):
        l = c // BT
        bt = c % BT
        return [
            pltpu.make_async_copy(
                stage.at[b, t],
                out.at[pl.ds(((l * NT + t) * BT + bt) * TILE, TILE)],
                sem_w.at[b])
            for t in range(NT)
        ]

    def outer(g, carry):
        for b in range(NBUF):
            c = c_base + g * NBUF + b

            @pl.when(g > 0)
            def _():
                for cp in write_copies(c, b):
                    cp.wait()

            for cp in idx_copies(c, b):
                cp.start()

        for b in range(NBUF):
            for cp in idx_copies(c_base + g * NBUF + b, b):
                cp.wait()
            for cp in gather_copies(b):
                cp.start()

        for b in range(NBUF):
            for cp in gather_copies(b):
                cp.wait()
            transpose(b)
            for cp in write_copies(c_base + g * NBUF + b, b):
                cp.start()

        return carry

    lax.fori_loop(0, PER_W // NBUF, outer, 0, unroll=False)

    for b in range(NBUF):
        for cp in write_copies(c_base, b):
            cp.wait()


def kernel(node_ids, cat_sector, cat_region, cat_venue,
           id_table, sector_table, region_table, venue_table):
    nid = node_ids.T.reshape(-1).astype(jnp.int32)
    sec = cat_sector.T.reshape(-1).astype(jnp.int32)
    reg = cat_region.T.reshape(-1).astype(jnp.int32)
    ven = cat_venue.T.reshape(-1).astype(jnp.int32)

    call = pl.kernel(
        _emb_body,
        out_type=jax.ShapeDtypeStruct((OUT_FLAT,), jnp.float32),
        mesh=plsc.VectorSubcoreMesh(
            core_axis_name="c", subcore_axis_name="s",
            num_cores=NC, num_subcores=NS),
        scratch_types=[
            pltpu.VMEM((NBUF, 4, CHUNK), jnp.int32),
            pltpu.VMEM((NBUF, CHUNK, D_ID), jnp.float32),
            pltpu.VMEM((NBUF, CHUNK, D_CAT), jnp.float32),
            pltpu.VMEM((NBUF, CHUNK, D_CAT), jnp.float32),
            pltpu.VMEM((NBUF, CHUNK, D_CAT), jnp.float32),
            pltpu.VMEM((NBUF, NT, TILE), jnp.float32),
            pltpu.SemaphoreType.DMA((NBUF,)),
            pltpu.SemaphoreType.DMA((NBUF,)),
            pltpu.SemaphoreType.DMA((NBUF,)),
        ],
        compiler_params=pltpu.CompilerParams(use_tc_tiling_on_sc=False,
                                             needs_layout_passes=False),
    )
    flat = call(nid, sec, reg, ven, id_table, sector_table, region_table,
                venue_table)
    return (flat.reshape(L, NT, BT, 8, CHUNK)
                .transpose(2, 4, 0, 1, 3)
                .reshape(B, L, D_OUT))


# speedup vs baseline: 1.0635x; 1.0019x over previous
import jax
import jax.numpy as jnp
from jax import lax
from jax.experimental import pallas as pl
from jax.experimental.pallas import tpu as pltpu
from jax.experimental.pallas import tpu_sc as plsc

NC, NS = 2, 16
NW = NC * NS
B, L = 16384, 20
R = B * L
CHUNK = 128
NBUF = 5
N_CHUNKS = L * (B // CHUNK)
PER_W = N_CHUNKS // NW

D_ID = 32
D_CAT = 16
D_OUT = D_ID + 3 * D_CAT
NT = D_OUT // 8
BT = B // CHUNK
TILE = 8 * CHUNK
OUT_FLAT = L * NT * BT * TILE


def _emb_body(nid, sec, reg, ven, id_t, sec_t, reg_t, ven_t, out,
              idx_v, id_r, sec_r, reg_r, ven_r, stage, sem_i, sem_g, sem_w):
    wid = lax.axis_index("s") * NC + lax.axis_index("c")
    c_base = wid * PER_W
    iota16 = lax.iota(jnp.int32, 16)

    def idx_off(c):
        l = c // BT
        bt = c % BT
        return l * B + bt * CHUNK

    def idx_copies(c, b):
        o = idx_off(c)
        return [
            pltpu.make_async_copy(nid.at[pl.ds(o, CHUNK)], idx_v.at[b, 0],
                                  sem_i.at[b]),
            pltpu.make_async_copy(sec.at[pl.ds(o, CHUNK)], idx_v.at[b, 1],
                                  sem_i.at[b]),
            pltpu.make_async_copy(reg.at[pl.ds(o, CHUNK)], idx_v.at[b, 2],
                                  sem_i.at[b]),
            pltpu.make_async_copy(ven.at[pl.ds(o, CHUNK)], idx_v.at[b, 3],
                                  sem_i.at[b]),
        ]

    def gather_copies(b):
        return [
            pltpu.make_async_copy(id_t.at[idx_v.at[b, 0]], id_r.at[b],
                                  sem_g.at[b]),
            pltpu.make_async_copy(sec_t.at[idx_v.at[b, 1]], sec_r.at[b],
                                  sem_g.at[b]),
            pltpu.make_async_copy(reg_t.at[idx_v.at[b, 2]], reg_r.at[b],
                                  sem_g.at[b]),
            pltpu.make_async_copy(ven_t.at[idx_v.at[b, 3]], ven_r.at[b],
                                  sem_g.at[b]),
        ]

    def tile_src(b, t):
        if t < 4:
            return id_r.at[b], t * 8
        if t < 6:
            return sec_r.at[b], (t - 4) * 8
        if t < 8:
            return reg_r.at[b], (t - 6) * 8
        return ven_r.at[b], (t - 8) * 8

    def transpose(b):
        def grp(j, carry):
            dr = j // 8
            br0 = (j % 8) * 16
            rows = br0 + iota16
            for t in range(NT):
                src, d0 = tile_src(b, t)
                cols = jnp.full((16,), d0 + dr, jnp.int32)
                stage[b, t, pl.ds(dr * CHUNK + br0, 16)] = (
                    plsc.load_gather(src, [rows, cols]))
            return carry

        lax.fori_loop(0, 64, grp, 0, unroll=2)

    def write_copies(c, b):
        l = c // BT
        bt = c % BT
        return [
            pltpu.make_async_copy(
                stage.at[b, t],
                out.at[pl.ds(((l * NT + t) * BT + bt) * TILE, TILE)],
                sem_w.at[b])
            for t in range(NT)
        ]

    def outer(g, carry):
        for b in range(NBUF):
            c = c_base + g * NBUF + b

            @pl.when(g > 0)
            def _():
                for cp in write_copies(c, b):
                    cp.wait()

            for cp in idx_copies(c, b):
                cp.start()

        for b in range(NBUF):
            for cp in idx_copies(c_base + g * NBUF + b, b):
                cp.wait()
            for cp in gather_copies(b):
                cp.start()

        for b in range(NBUF):
            for cp in gather_copies(b):
                cp.wait()
            transpose(b)
            for cp in write_copies(c_base + g * NBUF + b, b):
                cp.start()

        return carry

    lax.fori_loop(0, PER_W // NBUF, outer, 0, unroll=False)

    for b in range(NBUF):
        for cp in write_copies(c_base, b):
            cp.wait()


def kernel(node_ids, cat_sector, cat_region, cat_venue,
           id_table, sector_table, region_table, venue_table):
    nid = node_ids.T.reshape(-1).astype(jnp.int32)
    sec = cat_sector.T.reshape(-1).astype(jnp.int32)
    reg = cat_region.T.reshape(-1).astype(jnp.int32)
    ven = cat_venue.T.reshape(-1).astype(jnp.int32)

    call = pl.kernel(
        _emb_body,
        out_type=jax.ShapeDtypeStruct((OUT_FLAT,), jnp.float32),
        mesh=plsc.VectorSubcoreMesh(
            core_axis_name="c", subcore_axis_name="s",
            num_cores=NC, num_subcores=NS),
        scratch_types=[
            pltpu.VMEM((NBUF, 4, CHUNK), jnp.int32),
            pltpu.VMEM((NBUF, CHUNK, D_ID), jnp.float32),
            pltpu.VMEM((NBUF, CHUNK, D_CAT), jnp.float32),
            pltpu.VMEM((NBUF, CHUNK, D_CAT), jnp.float32),
            pltpu.VMEM((NBUF, CHUNK, D_CAT), jnp.float32),
            pltpu.VMEM((NBUF, NT, TILE), jnp.float32),
            pltpu.SemaphoreType.DMA((NBUF,)),
            pltpu.SemaphoreType.DMA((NBUF,)),
            pltpu.SemaphoreType.DMA((NBUF,)),
        ],
        compiler_params=pltpu.CompilerParams(use_tc_tiling_on_sc=False,
                                             needs_layout_passes=False),
    )
    flat = call(nid, sec, reg, ven, id_table, sector_table, region_table,
                venue_table)
    return (flat.reshape(L, NT, BT, 8, CHUNK)
                .transpose(2, 4, 0, 1, 3)
                .reshape(B, L, D_OUT))
